# baseline (speedup 1.0000x reference)
import dataclasses

import jax
import jax.numpy as jnp
from jax import lax
from jax.experimental import pallas as pl
from jax.experimental.pallas import tpu as pltpu
from jax.experimental.pallas import tpu_sc as plsc

N = 10000
D = 128
E = 320000
NC = 2
NS = 16
P = 10240
EP = 327680
ER = EP // 128
RPS = ER // NS
RPW = ER // (NC * NS)
SLICE = P // NS

_mesh = plsc.VectorSubcoreMesh(core_axis_name="c", subcore_axis_name="s")
_cp_no_layout = pltpu.CompilerParams()
if "needs_layout_passes" in pltpu.CompilerParams.__dataclass_fields__:
    _cp_no_layout = dataclasses.replace(_cp_no_layout, needs_layout_passes=False)


def _sc_degree(se):
    @pl.kernel(
        out_type=jax.ShapeDtypeStruct((2, P // 128, 128), jnp.float32),
        mesh=_mesh,
        compiler_params=_cp_no_layout,
        scratch_types=[
            pltpu.VMEM((RPS, 128), jnp.int32),
            pltpu.VMEM((P // 128, 128), jnp.float32),
            pltpu.VMEM_SHARED((P // 128, 128), jnp.float32),
            pltpu.SemaphoreType.DMA,
        ],
    )
    def deg_kernel(se_hbm, zer_hbm, out_hbm, idx_v, hist_v, deg_sh, sem):
        c = lax.axis_index("c")
        s = lax.axis_index("s")
        rows = P // 128
        pltpu.async_copy(zer_hbm.at[pl.ds(0, rows)], hist_v, sem).wait()

        @pl.when(s < rows // 8)
        def _():
            pltpu.sync_copy(zer_hbm.at[pl.ds(0, 8)], deg_sh.at[pl.ds(s * 8, 8)])
        pltpu.async_copy(se_hbm.at[c, pl.ds(s * RPS, RPS)], idx_v, sem).wait()
        ones16 = jnp.ones((16,), jnp.float32)

        @pl.loop(0, RPS)
        def _(r):
            @pl.loop(0, 128, step=16)
            def _(g):
                v = idx_v[r, pl.ds(g, 16)]
                plsc.addupdate_scatter(hist_v, [v >> 7, v & 127], ones16)

        plsc.subcore_barrier()
        iota16 = lax.iota(jnp.int32, 16)

        @pl.loop(0, rows, step=16)
        def _(g):
            pltpu.sync_copy(hist_v.at[pl.ds(g, 16)], deg_sh.at[iota16 + g],
                            add=True)

        plsc.subcore_barrier()

        @pl.when(s < rows // 8)
        def _():
            pltpu.sync_copy(
                deg_sh.at[pl.ds(s * 8, 8)],
                out_hbm.at[c, pl.ds(s * 8, 8)],
            )

    zer = jnp.zeros((128, 128), jnp.float32)
    return deg_kernel(se, zer)


def _sc_agg(hs, srcp, dstp):
    @pl.kernel(
        out_type=jax.ShapeDtypeStruct((2, P, 128), jnp.float32),
        mesh=_mesh,
        scratch_types=[
            pltpu.VMEM((RPW // 2, 128), jnp.int32),
            pltpu.VMEM((RPW // 2, 128), jnp.int32),
            pltpu.VMEM((128, 128), jnp.float32),
            pltpu.VMEM((128, 128), jnp.float32),
            pltpu.VMEM_SHARED((P, 128), jnp.float32),
            pltpu.SemaphoreType.DMA,
            pltpu.SemaphoreType.DMA,
            pltpu.SemaphoreType.DMA,
            pltpu.SemaphoreType.DMA,
            pltpu.SemaphoreType.DMA,
        ],
    )
    def agg_kernel(hs_hbm, src_hbm, dst_hbm, zer_hbm, out_hbm, src_v, dst_v,
                   buf_a, buf_b, agg_sh, sem, sem_a, sem_b, sem_sa, sem_sb):
        c = lax.axis_index("c")
        s = lax.axis_index("s")
        w = c * NS + s
        hrpw = RPW // 2
        for p in range(2):
            base = w * RPW + p * hrpw
            cp_s = pltpu.async_copy(src_hbm.at[pl.ds(base, hrpw)], src_v, sem)
            cp_d = pltpu.async_copy(dst_hbm.at[pl.ds(base, hrpw)], dst_v, sem)
            if p == 0:
                @pl.loop(0, SLICE, step=128)
                def _(r0):
                    pltpu.sync_copy(zer_hbm, agg_sh.at[pl.ds(s * SLICE + r0, 128)])

            cp_s.wait()
            cp_d.wait()
            pltpu.async_copy(hs_hbm.at[src_v.at[0]], buf_a, sem_a)
            pltpu.async_copy(hs_hbm.at[src_v.at[1]], buf_b, sem_b)
            if p == 0:
                plsc.subcore_barrier()

            @pl.loop(0, hrpw // 2)
            def _(j):
                i0 = 2 * j
                pltpu.make_async_copy(hs_hbm.at[src_v.at[i0]], buf_a, sem_a).wait()
                pltpu.sync_copy(buf_a, agg_sh.at[dst_v.at[i0]], add=True)

                @pl.when(j < hrpw // 2 - 1)
                def _():
                    pltpu.async_copy(hs_hbm.at[src_v.at[i0 + 2]], buf_a, sem_a)

                i1 = i0 + 1
                pltpu.make_async_copy(hs_hbm.at[src_v.at[i1]], buf_b, sem_b).wait()
                pltpu.sync_copy(buf_b, agg_sh.at[dst_v.at[i1]], add=True)

                @pl.when(j < hrpw // 2 - 1)
                def _():
                    pltpu.async_copy(hs_hbm.at[src_v.at[i1 + 2]], buf_b, sem_b)

        plsc.subcore_barrier()
        pltpu.sync_copy(
            agg_sh.at[pl.ds(s * SLICE, SLICE)],
            out_hbm.at[c, pl.ds(s * SLICE, SLICE)],
        )

    zer = jnp.zeros((128, 128), jnp.float32)
    return agg_kernel(hs, srcp, dstp, zer)


_R = 1024


def _tc_matmul_in(x_p, w_in, b_in):
    def body(x_ref, w_ref, b_ref, o_ref):
        h = jnp.dot(x_ref[...], w_ref[...], preferred_element_type=jnp.float32)
        o_ref[...] = h + b_ref[...]

    return pl.pallas_call(
        body,
        grid=(P // _R,),
        in_specs=[
            pl.BlockSpec((_R, D), lambda i: (i, 0)),
            pl.BlockSpec((D, D), lambda i: (0, 0)),
            pl.BlockSpec((1, D), lambda i: (0, 0)),
        ],
        out_specs=pl.BlockSpec((_R, D), lambda i: (i, 0)),
        out_shape=jax.ShapeDtypeStruct((P, D), jnp.float32),
    )(x_p, w_in, b_in)


def _tc_scale(h0, deg):
    def body(h_ref, dg_ref, o_ref):
        norm = lax.rsqrt(jnp.maximum(dg_ref[0, :, 0:1], 1.0))
        o_ref[...] = h_ref[...] * norm

    return pl.pallas_call(
        body,
        grid=(P // _R,),
        in_specs=[
            pl.BlockSpec((_R, D), lambda i: (i, 0)),
            pl.BlockSpec((1, _R, 1), lambda i: (0, i, 0)),
        ],
        out_specs=pl.BlockSpec((_R, D), lambda i: (i, 0)),
        out_shape=jax.ShapeDtypeStruct((P, D), jnp.float32),
    )(h0, deg)


def _tc_mid(agg, deg, w_conv, b_conv):
    def body(a0_ref, a1_ref, dgi_ref, dgo_ref, w_ref, b_ref, o_ref):
        ni = lax.rsqrt(jnp.maximum(dgi_ref[0, :, 0:1], 1.0))
        no = lax.rsqrt(jnp.maximum(dgo_ref[0, :, 0:1], 1.0))
        sagg = (a0_ref[0] + a1_ref[0]) * ni
        h = jnp.dot(sagg, w_ref[...], preferred_element_type=jnp.float32)
        h = jnp.maximum(h + b_ref[...], 0.0)
        o_ref[...] = h * no

    return pl.pallas_call(
        body,
        grid=(P // _R,),
        in_specs=[
            pl.BlockSpec((1, _R, D), lambda i: (0, i, 0)),
            pl.BlockSpec((1, _R, D), lambda i: (1, i, 0)),
            pl.BlockSpec((1, _R, 1), lambda i: (1, i, 0)),
            pl.BlockSpec((1, _R, 1), lambda i: (0, i, 0)),
            pl.BlockSpec((D, D), lambda i: (0, 0)),
            pl.BlockSpec((1, D), lambda i: (0, 0)),
        ],
        out_specs=pl.BlockSpec((_R, D), lambda i: (i, 0)),
        out_shape=jax.ShapeDtypeStruct((P, D), jnp.float32),
    )(agg, agg, deg, deg, w_conv, b_conv)


def _tc_last(agg, deg, w_conv, b_conv, w_out, b_out):
    grid = P // _R

    def body(a0_ref, a1_ref, dgi_ref, w_ref, b_ref, wo_ref, bo_ref, o_ref, acc):
        i = pl.program_id(0)
        ni = lax.rsqrt(jnp.maximum(dgi_ref[0, :, 0:1], 1.0))
        sagg = (a0_ref[0] + a1_ref[0]) * ni
        h = jnp.dot(sagg, w_ref[...], preferred_element_type=jnp.float32)
        h = jnp.maximum(h + b_ref[...], 0.0)
        rows = lax.broadcasted_iota(jnp.int32, (_R, 1), 0) + i * _R
        h = jnp.where(rows < N, h, 0.0)
        part = jnp.sum(h, axis=0, keepdims=True)

        @pl.when(i == 0)
        def _():
            acc[...] = part

        @pl.when(i > 0)
        def _():
            acc[...] = acc[...] + part

        @pl.when(i == grid - 1)
        def _():
            hg = acc[...] * (1.0 / N)
            o_ref[...] = (
                jnp.dot(hg, wo_ref[...], preferred_element_type=jnp.float32)
                + bo_ref[...]
            )

    return pl.pallas_call(
        body,
        grid=(grid,),
        in_specs=[
            pl.BlockSpec((1, _R, D), lambda i: (0, i, 0)),
            pl.BlockSpec((1, _R, D), lambda i: (1, i, 0)),
            pl.BlockSpec((1, _R, 1), lambda i: (1, i, 0)),
            pl.BlockSpec((D, D), lambda i: (0, 0)),
            pl.BlockSpec((1, D), lambda i: (0, 0)),
            pl.BlockSpec((D, D), lambda i: (0, 0)),
            pl.BlockSpec((1, D), lambda i: (0, 0)),
        ],
        out_specs=pl.BlockSpec((1, D), lambda i: (0, 0)),
        out_shape=jax.ShapeDtypeStruct((1, D), jnp.float32),
        scratch_shapes=[pltpu.VMEM((1, D), jnp.float32)],
    )(agg, agg, deg, w_conv, b_conv, w_out, b_out)


def kernel(x, edge_index, W_in, b_in, W_conv, b_conv, W_out, b_out):
    src = edge_index[0].astype(jnp.int32)
    dst = edge_index[1].astype(jnp.int32)
    padi = (jnp.arange(EP - E, dtype=jnp.int32) % (P - N)) + N
    srcp = jnp.concatenate([src, padi]).reshape(ER, 128)
    dstp = jnp.concatenate([dst, padi]).reshape(ER, 128)
    se = jnp.stack([srcp, dstp])
    x_p = jnp.concatenate([x, jnp.zeros((P - N, D), jnp.float32)], axis=0)
    b_in2 = b_in.reshape(1, D)
    b_conv2 = b_conv.reshape(1, D)
    b_out2 = b_out.reshape(1, D)

    deg = _sc_degree(se).reshape(2, P, 1)
    h0 = _tc_matmul_in(x_p, W_in, b_in2)
    hs0 = _tc_scale(h0, deg)
    agg1 = _sc_agg(hs0, srcp, dstp)
    hs1 = _tc_mid(agg1, deg, W_conv, b_conv2)
    agg2 = _sc_agg(hs1, srcp, dstp)
    return _tc_last(agg2, deg, W_conv, b_conv2, W_out, b_out2)


# speedup vs baseline: 1.0055x; 1.0055x over previous
import dataclasses

import jax
import jax.numpy as jnp
from jax import lax
from jax.experimental import pallas as pl
from jax.experimental.pallas import tpu as pltpu
from jax.experimental.pallas import tpu_sc as plsc

N = 10000
D = 128
E = 320000
NC = 2
NS = 16
P = 10240
EP = 327680
ER = EP // 128
RPS = ER // NS
RPW = ER // (NC * NS)
SLICE = P // NS

_mesh = plsc.VectorSubcoreMesh(core_axis_name="c", subcore_axis_name="s")
_cp_no_layout = pltpu.CompilerParams()
if "needs_layout_passes" in pltpu.CompilerParams.__dataclass_fields__:
    _cp_no_layout = dataclasses.replace(_cp_no_layout, needs_layout_passes=False)


def _sc_degree(se):
    @pl.kernel(
        out_type=jax.ShapeDtypeStruct((2, P // 128, 128), jnp.float32),
        mesh=_mesh,
        compiler_params=_cp_no_layout,
        scratch_types=[
            pltpu.VMEM((RPS, 128), jnp.int32),
            pltpu.VMEM((P // 128, 128), jnp.float32),
            pltpu.VMEM_SHARED((P // 128, 128), jnp.float32),
            pltpu.SemaphoreType.DMA,
        ],
    )
    def deg_kernel(se_hbm, zer_hbm, out_hbm, idx_v, hist_v, deg_sh, sem):
        c = lax.axis_index("c")
        s = lax.axis_index("s")
        rows = P // 128
        pltpu.async_copy(zer_hbm.at[pl.ds(0, rows)], hist_v, sem).wait()

        @pl.when(s < rows // 8)
        def _():
            pltpu.sync_copy(zer_hbm.at[pl.ds(0, 8)], deg_sh.at[pl.ds(s * 8, 8)])
        pltpu.async_copy(se_hbm.at[c, pl.ds(s * RPS, RPS)], idx_v, sem).wait()
        ones16 = jnp.ones((16,), jnp.float32)

        @pl.loop(0, RPS)
        def _(r):
            @pl.loop(0, 128, step=16)
            def _(g):
                v = idx_v[r, pl.ds(g, 16)]
                plsc.addupdate_scatter(hist_v, [v >> 7, v & 127], ones16)

        plsc.subcore_barrier()
        iota16 = lax.iota(jnp.int32, 16)

        @pl.loop(0, rows, step=16)
        def _(g):
            pltpu.sync_copy(hist_v.at[pl.ds(g, 16)], deg_sh.at[iota16 + g],
                            add=True)

        plsc.subcore_barrier()

        @pl.when(s < rows // 8)
        def _():
            pltpu.sync_copy(
                deg_sh.at[pl.ds(s * 8, 8)],
                out_hbm.at[c, pl.ds(s * 8, 8)],
            )

    zer = jnp.zeros((128, 128), jnp.float32)
    return deg_kernel(se, zer)


def _sc_agg(hs, srcp, dstp):
    @pl.kernel(
        out_type=jax.ShapeDtypeStruct((2, P, 128), jnp.float32),
        mesh=_mesh,
        scratch_types=[
            pltpu.VMEM((RPW // 2, 128), jnp.int32),
            pltpu.VMEM((RPW // 2, 128), jnp.int32),
            pltpu.VMEM((128, 128), jnp.float32),
            pltpu.VMEM((128, 128), jnp.float32),
            pltpu.VMEM_SHARED((P, 128), jnp.float32),
            pltpu.SemaphoreType.DMA,
            pltpu.SemaphoreType.DMA,
            pltpu.SemaphoreType.DMA,
            pltpu.SemaphoreType.DMA,
            pltpu.SemaphoreType.DMA,
        ],
    )
    def agg_kernel(hs_hbm, src_hbm, dst_hbm, zer_hbm, out_hbm, src_v, dst_v,
                   buf_a, buf_b, agg_sh, sem, sem_a, sem_b, sem_sa, sem_sb):
        c = lax.axis_index("c")
        s = lax.axis_index("s")
        w = c * NS + s
        hrpw = RPW // 2
        for p in range(2):
            base = w * RPW + p * hrpw
            cp_s = pltpu.async_copy(src_hbm.at[pl.ds(base, hrpw)], src_v, sem)
            cp_d = pltpu.async_copy(dst_hbm.at[pl.ds(base, hrpw)], dst_v, sem)
            if p == 0:
                @pl.loop(0, SLICE, step=128)
                def _(r0):
                    pltpu.sync_copy(zer_hbm, agg_sh.at[pl.ds(s * SLICE + r0, 128)])

            cp_s.wait()
            cp_d.wait()
            pltpu.async_copy(hs_hbm.at[src_v.at[0]], buf_a, sem_a)
            pltpu.async_copy(hs_hbm.at[src_v.at[1]], buf_b, sem_b)
            if p == 0:
                plsc.subcore_barrier()

            @pl.loop(0, hrpw // 2)
            def _(j):
                i0 = 2 * j
                pltpu.make_async_copy(hs_hbm.at[src_v.at[i0]], buf_a, sem_a).wait()
                pltpu.sync_copy(buf_a, agg_sh.at[dst_v.at[i0]], add=True)

                @pl.when(j < hrpw // 2 - 1)
                def _():
                    pltpu.async_copy(hs_hbm.at[src_v.at[i0 + 2]], buf_a, sem_a)

                i1 = i0 + 1
                pltpu.make_async_copy(hs_hbm.at[src_v.at[i1]], buf_b, sem_b).wait()
                pltpu.sync_copy(buf_b, agg_sh.at[dst_v.at[i1]], add=True)

                @pl.when(j < hrpw // 2 - 1)
                def _():
                    pltpu.async_copy(hs_hbm.at[src_v.at[i1 + 2]], buf_b, sem_b)

        plsc.subcore_barrier()
        pltpu.sync_copy(
            agg_sh.at[pl.ds(s * SLICE, SLICE)],
            out_hbm.at[c, pl.ds(s * SLICE, SLICE)],
        )

    zer = jnp.zeros((128, 128), jnp.float32)
    return agg_kernel(hs, srcp, dstp, zer)


_R = 1024


def _tc_first(x_p, deg, w_in, b_in):
    def body(x_ref, dg_ref, w_ref, b_ref, o_ref):
        norm = lax.rsqrt(jnp.maximum(dg_ref[0, :, 0:1], 1.0))
        h = jnp.dot(x_ref[...], w_ref[...], preferred_element_type=jnp.float32)
        o_ref[...] = (h + b_ref[...]) * norm

    return pl.pallas_call(
        body,
        grid=(P // _R,),
        in_specs=[
            pl.BlockSpec((_R, D), lambda i: (i, 0)),
            pl.BlockSpec((1, _R, 1), lambda i: (0, i, 0)),
            pl.BlockSpec((D, D), lambda i: (0, 0)),
            pl.BlockSpec((1, D), lambda i: (0, 0)),
        ],
        out_specs=pl.BlockSpec((_R, D), lambda i: (i, 0)),
        out_shape=jax.ShapeDtypeStruct((P, D), jnp.float32),
    )(x_p, deg, w_in, b_in)


def _tc_mid(agg, deg, w_conv, b_conv):
    def body(a0_ref, a1_ref, dgi_ref, dgo_ref, w_ref, b_ref, o_ref):
        ni = lax.rsqrt(jnp.maximum(dgi_ref[0, :, 0:1], 1.0))
        no = lax.rsqrt(jnp.maximum(dgo_ref[0, :, 0:1], 1.0))
        sagg = (a0_ref[0] + a1_ref[0]) * ni
        h = jnp.dot(sagg, w_ref[...], preferred_element_type=jnp.float32)
        h = jnp.maximum(h + b_ref[...], 0.0)
        o_ref[...] = h * no

    return pl.pallas_call(
        body,
        grid=(P // _R,),
        in_specs=[
            pl.BlockSpec((1, _R, D), lambda i: (0, i, 0)),
            pl.BlockSpec((1, _R, D), lambda i: (1, i, 0)),
            pl.BlockSpec((1, _R, 1), lambda i: (1, i, 0)),
            pl.BlockSpec((1, _R, 1), lambda i: (0, i, 0)),
            pl.BlockSpec((D, D), lambda i: (0, 0)),
            pl.BlockSpec((1, D), lambda i: (0, 0)),
        ],
        out_specs=pl.BlockSpec((_R, D), lambda i: (i, 0)),
        out_shape=jax.ShapeDtypeStruct((P, D), jnp.float32),
    )(agg, agg, deg, deg, w_conv, b_conv)


def _tc_last(agg, deg, w_conv, b_conv, w_out, b_out):
    grid = P // _R

    def body(a0_ref, a1_ref, dgi_ref, w_ref, b_ref, wo_ref, bo_ref, o_ref, acc):
        i = pl.program_id(0)
        ni = lax.rsqrt(jnp.maximum(dgi_ref[0, :, 0:1], 1.0))
        sagg = (a0_ref[0] + a1_ref[0]) * ni
        h = jnp.dot(sagg, w_ref[...], preferred_element_type=jnp.float32)
        h = jnp.maximum(h + b_ref[...], 0.0)
        rows = lax.broadcasted_iota(jnp.int32, (_R, 1), 0) + i * _R
        h = jnp.where(rows < N, h, 0.0)
        part = jnp.sum(h, axis=0, keepdims=True)

        @pl.when(i == 0)
        def _():
            acc[...] = part

        @pl.when(i > 0)
        def _():
            acc[...] = acc[...] + part

        @pl.when(i == grid - 1)
        def _():
            hg = acc[...] * (1.0 / N)
            o_ref[...] = (
                jnp.dot(hg, wo_ref[...], preferred_element_type=jnp.float32)
                + bo_ref[...]
            )

    return pl.pallas_call(
        body,
        grid=(grid,),
        in_specs=[
            pl.BlockSpec((1, _R, D), lambda i: (0, i, 0)),
            pl.BlockSpec((1, _R, D), lambda i: (1, i, 0)),
            pl.BlockSpec((1, _R, 1), lambda i: (1, i, 0)),
            pl.BlockSpec((D, D), lambda i: (0, 0)),
            pl.BlockSpec((1, D), lambda i: (0, 0)),
            pl.BlockSpec((D, D), lambda i: (0, 0)),
            pl.BlockSpec((1, D), lambda i: (0, 0)),
        ],
        out_specs=pl.BlockSpec((1, D), lambda i: (0, 0)),
        out_shape=jax.ShapeDtypeStruct((1, D), jnp.float32),
        scratch_shapes=[pltpu.VMEM((1, D), jnp.float32)],
    )(agg, agg, deg, w_conv, b_conv, w_out, b_out)


def kernel(x, edge_index, W_in, b_in, W_conv, b_conv, W_out, b_out):
    src = edge_index[0].astype(jnp.int32)
    dst = edge_index[1].astype(jnp.int32)
    padi = (jnp.arange(EP - E, dtype=jnp.int32) % (P - N)) + N
    srcp = jnp.concatenate([src, padi]).reshape(ER, 128)
    dstp = jnp.concatenate([dst, padi]).reshape(ER, 128)
    se = jnp.stack([srcp, dstp])
    x_p = jnp.concatenate([x, jnp.zeros((P - N, D), jnp.float32)], axis=0)
    b_in2 = b_in.reshape(1, D)
    b_conv2 = b_conv.reshape(1, D)
    b_out2 = b_out.reshape(1, D)

    deg = _sc_degree(se).reshape(2, P, 1)
    hs0 = _tc_first(x_p, deg, W_in, b_in2)
    agg1 = _sc_agg(hs0, srcp, dstp)
    hs1 = _tc_mid(agg1, deg, W_conv, b_conv2)
    agg2 = _sc_agg(hs1, srcp, dstp)
    return _tc_last(agg2, deg, W_conv, b_conv2, W_out, b_out2)
